# baseline (speedup 1.0000x reference)
import functools

import jax
import jax.numpy as jnp
from jax import lax
from jax.experimental import pallas as pl
from jax.experimental.pallas import tpu as pltpu
from jax.experimental.pallas import tpu_sc as plsc

N_PTS = 320000
N_SEG = 10000
D_FEAT = 128
NLANE = D_FEAT // 16

CPDMA = 2048
CP = CPDMA - 128
CPSTG = CPDMA + 16
CIDMA = CPDMA + 128
SDMA = 512
CX = 256
CXDMA = CX + 8
XIDMA = 512
IDX_PAD = 4096
NCHP = 8
NCHX = 56

_NEG_INF = float("-inf")
_SENTINEL = N_SEG + 999


def _rsqrt_f32(v):
    bits = lax.bitcast_convert_type(v, jnp.int32)
    y = lax.bitcast_convert_type(
        jnp.int32(0x5F3759DF) - (bits >> 1), jnp.float32)
    for _ in range(3):
        y = y * (1.5 - 0.5 * v * y * y)
    return y


def _ld(ref, i):
    return ref[pl.ds(i, 16)][0]


def _sc_body(spw, px, py, pz, xf, starts_h, idx_h, outx_h, out4_h,
             starts_v, spx, spy, spz, sidx, xstg, xidx, outx_v, out4_v,
             sumx_v, sumy_v, sumz_v, maxx_v, maxy_v, maxz_v,
             cenx_v, ceny_v, cenz_v, radr_v, dsem):
    nc = 2
    wid = lax.axis_index("s") * nc + lax.axis_index("c")
    seg_lo = pl.multiple_of(wid * spw, 8)

    lanes = lax.iota(jnp.int32, 16)
    zeros16 = jnp.zeros((16,), jnp.float32)
    ninf16 = jnp.full((16,), _NEG_INF, jnp.float32)

    sbase = pl.multiple_of(seg_lo & ~jnp.int32(127), 128)
    soff = seg_lo - sbase
    pltpu.async_copy(starts_h.at[pl.ds(sbase, SDMA)],
                     starts_v.at[pl.ds(0, SDMA)], dsem).wait()

    p0 = _ld(starts_v, soff)
    p1 = _ld(starts_v, soff + spw)

    def _init(s, _):
        s16 = pl.ds(s * 16, 16)
        sumx_v[s16] = zeros16
        sumy_v[s16] = zeros16
        sumz_v[s16] = zeros16
        maxx_v[s16] = ninf16
        maxy_v[s16] = ninf16
        maxz_v[s16] = ninf16
        for j in range(NLANE):
            outx_v[s, pl.ds(16 * j, 16)] = ninf16
        return 0
    lax.fori_loop(0, spw, _init, 0)

    m0_16 = lanes == 0

    @pl.loop(0, NCHP,
             init_carry=(zeros16, zeros16, zeros16, ninf16, ninf16, ninf16))
    def p1_chunk(k, carry):
        ax, ay, az, qx, qy, qz = carry
        base = p0 + k * CP
        nrows = jnp.minimum(CP, p1 - base)
        base_dma = pl.multiple_of(
            jnp.minimum(base & ~jnp.int32(127), N_PTS - CPDMA), 128)
        shift = base - base_dma
        pltpu.async_copy(px.at[pl.ds(base_dma, CPDMA)],
                         spx.at[pl.ds(0, CPDMA)], dsem).wait()
        pltpu.async_copy(py.at[pl.ds(base_dma, CPDMA)],
                         spy.at[pl.ds(0, CPDMA)], dsem).wait()
        pltpu.async_copy(pz.at[pl.ds(base_dma, CPDMA)],
                         spz.at[pl.ds(0, CPDMA)], dsem).wait()
        pltpu.async_copy(idx_h.at[pl.ds(base_dma, CIDMA)],
                         sidx.at[pl.ds(0, CIDMA)], dsem).wait()

        def row_body(r, c):
            ax, ay, az, qx, qy, qz = c
            valid = r < nrows
            q = jnp.minimum(r + shift, CPDMA)
            id0 = _ld(sidx, q)
            id1 = _ld(sidx, q + 1)
            vx = spx[pl.ds(q, 16)]
            vy = spy[pl.ds(q, 16)]
            vz = spz[pl.ds(q, 16)]
            ax = jnp.where(valid, ax + jnp.where(m0_16, vx, zeros16), ax)
            ay = jnp.where(valid, ay + jnp.where(m0_16, vy, zeros16), ay)
            az = jnp.where(valid, az + jnp.where(m0_16, vz, zeros16), az)
            qx = jnp.where(valid,
                           jnp.maximum(qx, jnp.where(m0_16, vx, ninf16)), qx)
            qy = jnp.where(valid,
                           jnp.maximum(qy, jnp.where(m0_16, vy, ninf16)), qy)
            qz = jnp.where(valid,
                           jnp.maximum(qz, jnp.where(m0_16, vz, ninf16)), qz)
            ends = valid & (id1 != id0)

            @pl.when(ends)
            def _flush():
                s16 = pl.ds((id0 - seg_lo) * 16, 16)
                sumx_v[s16] = sumx_v[s16] + ax
                sumy_v[s16] = sumy_v[s16] + ay
                sumz_v[s16] = sumz_v[s16] + az
                maxx_v[s16] = jnp.maximum(maxx_v[s16], qx)
                maxy_v[s16] = jnp.maximum(maxy_v[s16], qy)
                maxz_v[s16] = jnp.maximum(maxz_v[s16], qz)

            ax = jnp.where(ends, zeros16, ax)
            ay = jnp.where(ends, zeros16, ay)
            az = jnp.where(ends, zeros16, az)
            qx = jnp.where(ends, ninf16, qx)
            qy = jnp.where(ends, ninf16, qy)
            qz = jnp.where(ends, ninf16, qz)
            return ax, ay, az, qx, qy, qz

        return lax.fori_loop(0, CP, row_body, (ax, ay, az, qx, qy, qz))

    def cen_body(s, _):
        a = _ld(starts_v, soff + s)
        b = _ld(starts_v, soff + s + 1)
        rcp = 1.0 / jnp.maximum(
            jnp.full((16,), (b - a).astype(jnp.float32)), 1.0)
        cenx_v[pl.ds(s, 16)] = jnp.full((16,), _ld(sumx_v, s * 16)) * rcp
        ceny_v[pl.ds(s, 16)] = jnp.full((16,), _ld(sumy_v, s * 16)) * rcp
        cenz_v[pl.ds(s, 16)] = jnp.full((16,), _ld(sumz_v, s * 16)) * rcp
        return 0
    lax.fori_loop(0, spw, cen_body, 0)

    def _rinit(s, _):
        sumx_v[pl.ds(s * 16, 16)] = ninf16
        return 0
    lax.fori_loop(0, spw, _rinit, 0)

    @pl.loop(0, NCHP, init_carry=ninf16)
    def p2_chunk(k, mr):
        base = p0 + k * CP
        nrows = jnp.minimum(CP, p1 - base)
        base_dma = pl.multiple_of(
            jnp.minimum(base & ~jnp.int32(127), N_PTS - CPDMA), 128)
        shift = base - base_dma
        pltpu.async_copy(px.at[pl.ds(base_dma, CPDMA)],
                         spx.at[pl.ds(0, CPDMA)], dsem).wait()
        pltpu.async_copy(py.at[pl.ds(base_dma, CPDMA)],
                         spy.at[pl.ds(0, CPDMA)], dsem).wait()
        pltpu.async_copy(pz.at[pl.ds(base_dma, CPDMA)],
                         spz.at[pl.ds(0, CPDMA)], dsem).wait()
        pltpu.async_copy(idx_h.at[pl.ds(base_dma, CIDMA)],
                         sidx.at[pl.ds(0, CIDMA)], dsem).wait()

        def row_body(r, mr):
            valid = r < nrows
            q = jnp.minimum(r + shift, CPDMA)
            id0 = _ld(sidx, q)
            id1 = _ld(sidx, q + 1)
            sl = jnp.minimum(jnp.maximum(id0 - seg_lo, 0), spw - 1)
            cx = _ld(cenx_v, sl)
            cy = _ld(ceny_v, sl)
            cz = _ld(cenz_v, sl)
            dx = spx[pl.ds(q, 16)] - cx
            dy = spy[pl.ds(q, 16)] - cy
            dz = spz[pl.ds(q, 16)] - cz
            r2 = dx * dx + dy * dy + dz * dz
            mr = jnp.where(
                valid, jnp.maximum(mr, jnp.where(m0_16, r2, ninf16)), mr)
            ends = valid & (id1 != id0)

            @pl.when(ends)
            def _flush():
                s16 = pl.ds(sl * 16, 16)
                sumx_v[s16] = jnp.maximum(sumx_v[s16], mr)

            return jnp.where(ends, ninf16, mr)

        return lax.fori_loop(0, CP, row_body, mr)

    def rad_body(s, _):
        a = _ld(starts_v, soff + s)
        b = _ld(starts_v, soff + s + 1)
        ne = b > a
        r2 = jnp.full((16,), _ld(sumx_v, s * 16))
        r2 = jnp.where(ne, r2, 1.0) + 1e-12
        rad = jnp.where(ne, r2 * _rsqrt_f32(r2), 0.0)
        radr_v[pl.ds(s, 16)] = rad
        maxx_v[pl.ds(s, 16)] = jnp.full((16,), _ld(maxx_v, s * 16))
        maxy_v[pl.ds(s, 16)] = jnp.full((16,), _ld(maxy_v, s * 16))
        maxz_v[pl.ds(s, 16)] = jnp.full((16,), _ld(maxz_v, s * 16))
        return 0
    lax.fori_loop(0, spw, rad_body, 0)

    def out4_body(k, _):
        o = pl.ds(16 * k, 16)
        rad = radr_v[o]
        ne = rad > 0.0
        diam = 2.0 * rad
        inv = 1.0 / (diam + 1e-8)
        k16 = 16 * k
        out4_v[pl.ds(k16, 16)] = jnp.where(
            ne, (maxx_v[o] - cenx_v[o]) * inv, zeros16)
        out4_v[pl.ds(spw + k16, 16)] = jnp.where(
            ne, (maxy_v[o] - ceny_v[o]) * inv, zeros16)
        out4_v[pl.ds(2 * spw + k16, 16)] = jnp.where(
            ne, (maxz_v[o] - cenz_v[o]) * inv, zeros16)
        out4_v[pl.ds(3 * spw + k16, 16)] = diam
        return 0
    lax.fori_loop(0, spw // 16, out4_body, 0)

    @pl.loop(0, NCHX,
             init_carry=tuple(ninf16 for _ in range(NLANE)))
    def p3_chunk(k, accs):
        base = p0 + k * CX
        nrows = jnp.minimum(CX, p1 - base)
        base_dma = pl.multiple_of(
            jnp.minimum(base & ~jnp.int32(7), N_PTS - CXDMA), 8)
        shift = base - base_dma
        ibase = pl.multiple_of(
            jnp.minimum(base & ~jnp.int32(127), N_PTS + IDX_PAD - XIDMA), 128)
        ishift = base - ibase
        pltpu.async_copy(xf.at[pl.ds(base_dma, CXDMA), :], xstg, dsem).wait()
        pltpu.async_copy(idx_h.at[pl.ds(ibase, XIDMA)],
                         xidx.at[pl.ds(0, XIDMA)], dsem).wait()

        def row_body(r, accs):
            valid = r < nrows
            iq = r + ishift
            id0 = _ld(xidx, iq)
            id1 = _ld(xidx, iq + 1)
            rr = jnp.minimum(r + shift, CXDMA - 1)
            accs = tuple(
                jnp.where(valid,
                          jnp.maximum(accs[j], xstg[rr, pl.ds(16 * j, 16)]),
                          accs[j])
                for j in range(NLANE))
            ends = valid & (id1 != id0)

            @pl.when(ends)
            def _flush():
                sl = id0 - seg_lo
                for j in range(NLANE):
                    oj = pl.ds(16 * j, 16)
                    outx_v[sl, oj] = jnp.maximum(outx_v[sl, oj], accs[j])

            return tuple(jnp.where(ends, ninf16, a) for a in accs)

        return lax.fori_loop(0, CX, row_body, accs)

    def fin_body(s, _):
        a = _ld(starts_v, soff + s)
        b = _ld(starts_v, soff + s + 1)
        ne = b > a
        for j in range(NLANE):
            oj = pl.ds(16 * j, 16)
            outx_v[s, oj] = jnp.where(ne, outx_v[s, oj], zeros16)
        return 0
    lax.fori_loop(0, spw, fin_body, 0)

    pltpu.async_copy(outx_v, outx_h.at[pl.ds(seg_lo, spw), :], dsem).wait()
    pltpu.async_copy(
        out4_v,
        out4_h.at[pl.ds(pl.multiple_of(wid * 4 * spw, 8), 4 * spw)],
        dsem).wait()


def kernel(pos, x, idx):
    info = plsc.get_sparse_core_info()
    nw = info.num_cores * info.num_subcores
    spw = -(-N_SEG // nw)
    spw = -(-spw // 16) * 16
    nseg_pad = spw * nw

    seg_ids = jnp.arange(nseg_pad + 1, dtype=jnp.int32)
    starts = jnp.searchsorted(idx, seg_ids, side="left").astype(jnp.int32)
    pad = (nseg_pad - 128) // 128 * 128 + SDMA + 128 - (nseg_pad + 1)
    starts = jnp.concatenate(
        [starts, jnp.full((pad,), N_PTS, jnp.int32)])
    idxp = jnp.concatenate(
        [idx, jnp.full((IDX_PAD,), _SENTINEL, jnp.int32)])

    px = pos[:, 0]
    py = pos[:, 1]
    pz = pos[:, 2]

    mesh = plsc.VectorSubcoreMesh(core_axis_name="c", subcore_axis_name="s")
    kfn = pl.kernel(
        functools.partial(_sc_body, spw),
        out_type=(
            jax.ShapeDtypeStruct((nseg_pad, D_FEAT), jnp.float32),
            jax.ShapeDtypeStruct((nw * 4 * spw,), jnp.float32),
        ),
        mesh=mesh,
        scratch_types=[
            pltpu.VMEM((SDMA + 16,), jnp.int32),
            pltpu.VMEM((CPSTG,), jnp.float32),
            pltpu.VMEM((CPSTG,), jnp.float32),
            pltpu.VMEM((CPSTG,), jnp.float32),
            pltpu.VMEM((CIDMA + 16,), jnp.int32),
            pltpu.VMEM((CXDMA, D_FEAT), jnp.float32),
            pltpu.VMEM((XIDMA + 16,), jnp.int32),
            pltpu.VMEM((spw, D_FEAT), jnp.float32),
            pltpu.VMEM((4 * spw,), jnp.float32),
            pltpu.VMEM((spw * 16,), jnp.float32),
            pltpu.VMEM((spw * 16,), jnp.float32),
            pltpu.VMEM((spw * 16,), jnp.float32),
            pltpu.VMEM((spw * 16,), jnp.float32),
            pltpu.VMEM((spw * 16,), jnp.float32),
            pltpu.VMEM((spw * 16,), jnp.float32),
            pltpu.VMEM((spw + 16,), jnp.float32),
            pltpu.VMEM((spw + 16,), jnp.float32),
            pltpu.VMEM((spw + 16,), jnp.float32),
            pltpu.VMEM((spw + 16,), jnp.float32),
            pltpu.SemaphoreType.DMA,
        ],
    )
    outx, out4f = kfn(px, py, pz, x, starts, idxp)
    out4 = out4f.reshape(nw, 4, spw).transpose(1, 0, 2).reshape(4, nseg_pad)
    x_global = jnp.concatenate(
        [outx[:N_SEG],
         out4[0:3, :N_SEG].T,
         out4[3:4, :N_SEG].T], axis=1)
    return x_global
